# baseline (speedup 1.0000x reference)
import jax
import jax.numpy as jnp
from jax import lax
from jax.experimental import pallas as pl
from jax.experimental.pallas import tpu as pltpu
from jax.experimental.pallas import tpu_sc as plsc

NUM_NODES = 10000
NUM_EDGES = 320000
NUM_RELATIONS = 500
EMBED_DIM = 128

NC = 2
NS = 16
NW = NC * NS
LANES = 16

EDGES_PER_W = NUM_EDGES // NW
CHUNK = 40
CHUNKS_PER_W = EDGES_PER_W // CHUNK
DSLICES = EMBED_DIM // LANES


def _dist_mult_body(z_hbm, src_hbm, dst_hbm, typ_hbm, rel_hbm, out_hbm,
                    out_all, tmp,
                    is0, id0, ir0, rs0, rd0, rr0,
                    is1, id1, ir1, rs1, rd1, rr1,
                    z_sh, rel_sh, sem0, sem1):
    sid = lax.axis_index("s")
    wid = sid * NC + lax.axis_index("c")
    base_w = wid * EDGES_PER_W
    iota16 = lax.iota(jnp.int32, LANES)
    slots = ((is0, id0, ir0, rs0, rd0, rr0, sem0),
             (is1, id1, ir1, rs1, rd1, rr1, sem1))

    @pl.when(sid == 0)
    def _stage():
        pltpu.sync_copy(z_hbm, z_sh)
        pltpu.sync_copy(rel_hbm, rel_sh)

    plsc.subcore_barrier()

    def fire(c, slot):
        isx, idx, irx, rs, rd, rr, sem = slots[slot]
        base = base_w + c * CHUNK
        pltpu.sync_copy(src_hbm.at[pl.ds(base, CHUNK)], isx)
        pltpu.sync_copy(dst_hbm.at[pl.ds(base, CHUNK)], idx)
        pltpu.sync_copy(typ_hbm.at[pl.ds(base, CHUNK)], irx)
        pltpu.async_copy(z_sh.at[isx], rs, sem)
        pltpu.async_copy(z_sh.at[idx], rd, sem)
        pltpu.async_copy(rel_sh.at[irx], rr, sem)

    def drain(slot):
        isx, idx, irx, rs, rd, rr, sem = slots[slot]
        pltpu.make_async_copy(z_sh.at[isx], rs, sem).wait()
        pltpu.make_async_copy(z_sh.at[idx], rd, sem).wait()
        pltpu.make_async_copy(rel_sh.at[irx], rr, sem).wait()

    def compute(c, slot):
        rows_s, rows_d, rows_r = slots[slot][3:6]

        def do_group(start, nel, out_off):
            for el in range(nel):
                e = start + el
                acc = (rows_s[e, pl.ds(0, LANES)]
                       * rows_r[e, pl.ds(0, LANES)]
                       * rows_d[e, pl.ds(0, LANES)])
                for j in range(1, DSLICES):
                    sl = pl.ds(j * LANES, LANES)
                    acc = acc + rows_s[e, sl] * rows_r[e, sl] * rows_d[e, sl]
                tmp[pl.ds(el * LANES, LANES)] = acc
            for s in (8, 4, 2, 1):
                for el in range(nel):
                    b = el * LANES
                    tmp[pl.ds(b, LANES)] = (
                        tmp[pl.ds(b, LANES)] + tmp[pl.ds(b + s, LANES)])
            out16 = jnp.zeros((LANES,), jnp.float32)
            for el in range(nel):
                w = tmp[pl.ds(el * (LANES - 1), LANES)]
                out16 = jnp.where(iota16 == el, w, out16)
            out_all[pl.ds(out_off, LANES)] = out16

        def group_body(g, _):
            do_group(g * LANES, LANES, c * CHUNK + g * LANES)
            return 0

        lax.fori_loop(0, 2, group_body, 0)
        do_group(32, 8, c * CHUNK + 32)

    fire(0, 0)

    def pair_body(h, _):
        c = 2 * h
        fire(c + 1, 1)
        drain(0)
        compute(c, 0)
        fire(c + 2, 0)
        drain(1)
        compute(c + 1, 1)
        return 0

    lax.fori_loop(0, (CHUNKS_PER_W - 2) // 2, pair_body, 0)
    fire(CHUNKS_PER_W - 1, 1)
    drain(0)
    compute(CHUNKS_PER_W - 2, 0)
    drain(1)
    compute(CHUNKS_PER_W - 1, 1)

    pltpu.sync_copy(out_all.at[pl.ds(0, EDGES_PER_W)],
                    out_hbm.at[pl.ds(base_w, EDGES_PER_W)])


@jax.jit
def kernel(z, edge_index, edge_type, rel_emb):
    src = edge_index[0].astype(jnp.int32)
    dst = edge_index[1].astype(jnp.int32)
    typ = edge_type.astype(jnp.int32)
    mesh = plsc.VectorSubcoreMesh(core_axis_name="c", subcore_axis_name="s")
    chunk_bufs = [
        pltpu.VMEM((CHUNK,), jnp.int32),
        pltpu.VMEM((CHUNK,), jnp.int32),
        pltpu.VMEM((CHUNK,), jnp.int32),
        pltpu.VMEM((CHUNK, EMBED_DIM), jnp.float32),
        pltpu.VMEM((CHUNK, EMBED_DIM), jnp.float32),
        pltpu.VMEM((CHUNK, EMBED_DIM), jnp.float32),
    ]
    k = pl.kernel(
        _dist_mult_body,
        out_type=jax.ShapeDtypeStruct((NUM_EDGES,), jnp.float32),
        mesh=mesh,
        scratch_types=[
            pltpu.VMEM((EDGES_PER_W + LANES,), jnp.float32),
            pltpu.VMEM((LANES * LANES + LANES,), jnp.float32),
            *chunk_bufs,
            *chunk_bufs,
            pltpu.VMEM_SHARED((NUM_NODES, EMBED_DIM), jnp.float32),
            pltpu.VMEM_SHARED((NUM_RELATIONS, EMBED_DIM), jnp.float32),
            pltpu.SemaphoreType.DMA,
            pltpu.SemaphoreType.DMA,
        ],
    )
    return k(z, src, dst, typ, rel_emb)


# speedup vs baseline: 2.4357x; 2.4357x over previous
import jax
import jax.numpy as jnp
from jax import lax
from jax.experimental import pallas as pl
from jax.experimental.pallas import tpu as pltpu
from jax.experimental.pallas import tpu_sc as plsc

NUM_NODES = 10000
NUM_EDGES = 320000
NUM_RELATIONS = 500
EMBED_DIM = 128

NC = 2
NS = 16
NW = NC * NS
LANES = 16

EDGES_PER_W = NUM_EDGES // NW
CHUNK = 80
CHUNKS_PER_W = EDGES_PER_W // CHUNK
DSLICES = EMBED_DIM // LANES
GROUPS = CHUNK // LANES


def _dist_mult_body(z_hbm, src_hbm, dst_hbm, typ_hbm, rel_hbm, out_hbm,
                    idx_s, idx_d, idx_r, out_all, tmp,
                    rs0, rd0, rr0, rs1, rd1, rr1, sem0, sem1):
    wid = lax.axis_index("s") * NC + lax.axis_index("c")
    base_w = wid * EDGES_PER_W
    iota16 = lax.iota(jnp.int32, LANES)
    slots = ((rs0, rd0, rr0, sem0), (rs1, rd1, rr1, sem1))

    pltpu.sync_copy(src_hbm.at[pl.ds(base_w, EDGES_PER_W)], idx_s)
    pltpu.sync_copy(dst_hbm.at[pl.ds(base_w, EDGES_PER_W)], idx_d)
    pltpu.sync_copy(typ_hbm.at[pl.ds(base_w, EDGES_PER_W)], idx_r)

    def fire(c, slot):
        rs, rd, rr, sem = slots[slot]
        off = c * CHUNK
        pltpu.async_copy(z_hbm.at[idx_s.at[pl.ds(off, CHUNK)]], rs, sem)
        pltpu.async_copy(z_hbm.at[idx_d.at[pl.ds(off, CHUNK)]], rd, sem)
        pltpu.async_copy(rel_hbm.at[idx_r.at[pl.ds(off, CHUNK)]], rr, sem)

    def drain(slot):
        rs, rd, rr, sem = slots[slot]
        pltpu.make_async_copy(z_hbm.at[idx_s.at[pl.ds(0, CHUNK)]], rs, sem).wait()
        pltpu.make_async_copy(z_hbm.at[idx_d.at[pl.ds(0, CHUNK)]], rd, sem).wait()
        pltpu.make_async_copy(rel_hbm.at[idx_r.at[pl.ds(0, CHUNK)]], rr, sem).wait()

    def compute(c, slot):
        rows_s, rows_d, rows_r, _ = slots[slot]

        def group_body(g, _):
            for el in range(LANES):
                e = g * LANES + el
                acc = (rows_s[e, pl.ds(0, LANES)]
                       * rows_r[e, pl.ds(0, LANES)]
                       * rows_d[e, pl.ds(0, LANES)])
                for j in range(1, DSLICES):
                    sl = pl.ds(j * LANES, LANES)
                    acc = acc + rows_s[e, sl] * rows_r[e, sl] * rows_d[e, sl]
                tmp[pl.ds(el * LANES, LANES)] = acc
            for s in (8, 4, 2, 1):
                for el in range(LANES):
                    b = el * LANES
                    tmp[pl.ds(b, LANES)] = (
                        tmp[pl.ds(b, LANES)] + tmp[pl.ds(b + s, LANES)])
            out16 = jnp.zeros((LANES,), jnp.float32)
            for el in range(LANES):
                w = tmp[pl.ds(el * (LANES - 1), LANES)]
                out16 = jnp.where(iota16 == el, w, out16)
            out_all[pl.ds(c * CHUNK + g * LANES, LANES)] = out16
            return 0

        lax.fori_loop(0, GROUPS, group_body, 0)

    fire(0, 0)

    def pair_body(h, _):
        c = 2 * h
        fire(c + 1, 1)
        drain(0)
        compute(c, 0)
        fire(c + 2, 0)
        drain(1)
        compute(c + 1, 1)
        return 0

    lax.fori_loop(0, (CHUNKS_PER_W - 1) // 2, pair_body, 0)
    drain(0)
    compute(CHUNKS_PER_W - 1, 0)

    pltpu.sync_copy(out_all, out_hbm.at[pl.ds(base_w, EDGES_PER_W)])


@jax.jit
def kernel(z, edge_index, edge_type, rel_emb):
    src = edge_index[0].astype(jnp.int32)
    dst = edge_index[1].astype(jnp.int32)
    typ = edge_type.astype(jnp.int32)
    mesh = plsc.VectorSubcoreMesh(core_axis_name="c", subcore_axis_name="s")
    k = pl.kernel(
        _dist_mult_body,
        out_type=jax.ShapeDtypeStruct((NUM_EDGES,), jnp.float32),
        mesh=mesh,
        scratch_types=[
            pltpu.VMEM((EDGES_PER_W,), jnp.int32),
            pltpu.VMEM((EDGES_PER_W,), jnp.int32),
            pltpu.VMEM((EDGES_PER_W,), jnp.int32),
            pltpu.VMEM((EDGES_PER_W,), jnp.float32),
            pltpu.VMEM((LANES * LANES + LANES,), jnp.float32),
            pltpu.VMEM((CHUNK, EMBED_DIM), jnp.float32),
            pltpu.VMEM((CHUNK, EMBED_DIM), jnp.float32),
            pltpu.VMEM((CHUNK, EMBED_DIM), jnp.float32),
            pltpu.VMEM((CHUNK, EMBED_DIM), jnp.float32),
            pltpu.VMEM((CHUNK, EMBED_DIM), jnp.float32),
            pltpu.VMEM((CHUNK, EMBED_DIM), jnp.float32),
            pltpu.SemaphoreType.DMA,
            pltpu.SemaphoreType.DMA,
        ],
    )
    return k(z, src, dst, typ, rel_emb)
